# baseline (speedup 1.0000x reference)
import functools

import jax
import jax.numpy as jnp
from jax.experimental import pallas as pl
from jax.experimental.pallas import tpu as pltpu
from jax.experimental.pallas import tpu_sc as plsc

BETA = 5.5
C = 1000
S = 32
D = 1024
CS = C * S
B = 4096
LANES = 16
SENTINEL = -3.0


def _matvec_body(m_ref, q_ref, o_ref):
    m = m_ref[...]
    q = q_ref[...]
    o_ref[...] = jnp.sum(m * q, axis=1)


def _matvec(x, q, block_rows):
    n = x.shape[0]
    return pl.pallas_call(
        _matvec_body,
        grid=(pl.cdiv(n, block_rows),),
        in_specs=[
            pl.BlockSpec((block_rows, D), lambda i: (i, 0)),
            pl.BlockSpec((1, D), lambda i: (0, 0)),
        ],
        out_specs=pl.BlockSpec((block_rows,), lambda i: (i,)),
        out_shape=jax.ShapeDtypeStruct((n,), jnp.float32),
    )(x, q)


@functools.cache
def _make_sc_scatter():
    mesh = plsc.VectorSubcoreMesh(core_axis_name="c", subcore_axis_name="s", num_cores=1)

    @functools.partial(
        pl.kernel,
        out_type=jax.ShapeDtypeStruct((CS,), jnp.float32),
        mesh=mesh,
        scratch_types=[
            pltpu.VMEM((CS,), jnp.float32),
            pltpu.VMEM((B,), jnp.int32),
            pltpu.VMEM((B,), jnp.float32),
        ],
        compiler_params=pltpu.CompilerParams(needs_layout_passes=False),
    )
    def _sc_scatter(idx_hbm, sval_hbm, out_hbm, s_v, idx_v, val_v):
        cid = jax.lax.axis_index("c")
        sid = jax.lax.axis_index("s")

        @pl.when(jnp.logical_and(cid == 0, sid == 0))
        def _():
            pltpu.sync_copy(idx_hbm, idx_v)
            pltpu.sync_copy(sval_hbm, val_v)

            sent = jnp.full((LANES,), SENTINEL, jnp.float32)

            def fill(k, carry):
                s_v[pl.ds(k * LANES, LANES)] = sent
                return carry

            jax.lax.fori_loop(0, CS // LANES, fill, 0)

            def step(k, carry):
                iv = idx_v[pl.ds(k * LANES, LANES)]
                vv = val_v[pl.ds(k * LANES, LANES)]
                plsc.store_scatter(s_v, [iv], vv)
                return carry

            jax.lax.fori_loop(0, B // LANES, step, 0)
            pltpu.sync_copy(s_v, out_hbm)

    return _sc_scatter


def _reduce_body(s_ref, w_ref, o_ref):
    s = s_ref[...]
    wv = w_ref[...]
    m = jnp.where(wv > -2.0, wv, s)
    w = jnp.exp((-BETA) * (1.0 - m))
    f = w * m
    seg_row = jax.lax.broadcasted_iota(jnp.int32, (128, 4), 0) // S
    seg_col = jax.lax.broadcasted_iota(jnp.int32, (128, 4), 1)
    seg = (seg_row == seg_col).astype(jnp.float32)
    o_ref[...] = jax.lax.dot_general(
        f, seg, (((1,), (0,)), ((), ())), preferred_element_type=jnp.float32)


def _reduce(s2, w2):
    return pl.pallas_call(
        _reduce_body,
        out_shape=jax.ShapeDtypeStruct((CS // 128, 4), jnp.float32),
    )(s2, w2)


def kernel(mem, idx, val, img_feat):
    q = img_feat.astype(jnp.float32)
    s_val = _matvec(val, q, 512)
    written = _make_sc_scatter()(idx.astype(jnp.int32),
                                 s_val)
    s_mem = _matvec(mem, q, 1024)
    logits4 = _reduce(s_mem.reshape(CS // 128, 128),
                      written.reshape(CS // 128, 128))
    return logits4.reshape(C)


# speedup vs baseline: 1.1585x; 1.1585x over previous
import functools

import jax
import jax.numpy as jnp
from jax.experimental import pallas as pl
from jax.experimental.pallas import tpu as pltpu
from jax.experimental.pallas import tpu_sc as plsc

BETA = 5.5
C = 1000
S = 32
D = 1024
CS = C * S
B = 4096
LANES = 16
SENTINEL = -3.0


def _matvec_body(m_ref, q_ref, o_ref):
    m = m_ref[...]
    q = q_ref[...]
    o_ref[...] = jnp.sum(m * q, axis=1)


def _matvec(x, q, block_rows):
    n = x.shape[0]
    return pl.pallas_call(
        _matvec_body,
        grid=(pl.cdiv(n, block_rows),),
        in_specs=[
            pl.BlockSpec((block_rows, D), lambda i: (i, 0)),
            pl.BlockSpec((1, D), lambda i: (0, 0)),
        ],
        out_specs=pl.BlockSpec((block_rows,), lambda i: (i,)),
        out_shape=jax.ShapeDtypeStruct((n,), jnp.float32),
    )(x, q)


@functools.cache
def _make_sc_scatter():
    mesh = plsc.VectorSubcoreMesh(core_axis_name="c", subcore_axis_name="s", num_cores=1)

    @functools.partial(
        pl.kernel,
        out_type=jax.ShapeDtypeStruct((CS,), jnp.float32),
        mesh=mesh,
        scratch_types=[
            pltpu.VMEM((CS,), jnp.float32),
            pltpu.VMEM((B,), jnp.int32),
            pltpu.VMEM((B,), jnp.float32),
        ],
        compiler_params=pltpu.CompilerParams(needs_layout_passes=False),
    )
    def _sc_scatter(idx_hbm, sval_hbm, out_hbm, s_v, idx_v, val_v):
        cid = jax.lax.axis_index("c")
        sid = jax.lax.axis_index("s")

        @pl.when(jnp.logical_and(cid == 0, sid == 0))
        def _():
            pltpu.sync_copy(idx_hbm, idx_v)
            pltpu.sync_copy(sval_hbm, val_v)

            sent = jnp.full((LANES,), SENTINEL, jnp.float32)

            def fill(k, carry):
                s_v[pl.ds(k * LANES, LANES)] = sent
                return carry

            jax.lax.fori_loop(0, CS // LANES, fill, 0)

            def step(k, carry):
                iv = idx_v[pl.ds(k * LANES, LANES)]
                vv = val_v[pl.ds(k * LANES, LANES)]
                plsc.store_scatter(s_v, [iv], vv)
                return carry

            jax.lax.fori_loop(0, B // LANES, step, 0)
            pltpu.sync_copy(s_v, out_hbm)

    return _sc_scatter


def _reduce_body(s_ref, w_ref, o_ref):
    s = s_ref[...]
    wv = w_ref[...]
    m = jnp.where(wv > -2.0, wv, s)
    w = jnp.exp((-BETA) * (1.0 - m))
    f = w * m
    seg_row = jax.lax.broadcasted_iota(jnp.int32, (128, 4), 0) // S
    seg_col = jax.lax.broadcasted_iota(jnp.int32, (128, 4), 1)
    seg = (seg_row == seg_col).astype(jnp.float32)
    o_ref[...] = jax.lax.dot_general(
        f, seg, (((1,), (0,)), ((), ())), preferred_element_type=jnp.float32)


def _reduce(s2, w2):
    return pl.pallas_call(
        _reduce_body,
        out_shape=jax.ShapeDtypeStruct((CS // 128, 4), jnp.float32),
    )(s2, w2)


def kernel(mem, idx, val, img_feat):
    q = img_feat.astype(jnp.float32)
    s_val = _matvec(val, q, 2048)
    written = _make_sc_scatter()(idx.astype(jnp.int32),
                                 s_val)
    s_mem = _matvec(mem, q, 2048)
    logits4 = _reduce(s_mem.reshape(CS // 128, 128),
                      written.reshape(CS // 128, 128))
    return logits4.reshape(C)


# speedup vs baseline: 1.1617x; 1.0028x over previous
import functools

import jax
import jax.numpy as jnp
from jax.experimental import pallas as pl
from jax.experimental.pallas import tpu as pltpu
from jax.experimental.pallas import tpu_sc as plsc

BETA = 5.5
C = 1000
S = 32
D = 1024
CS = C * S
B = 4096
LANES = 16
SENTINEL = -3.0


def _matvec_body(m_ref, q_ref, o_ref):
    m = m_ref[...]
    q = q_ref[...]
    o_ref[...] = jnp.sum(m * q, axis=1)


def _matvec(x, q, block_rows):
    n = x.shape[0]
    return pl.pallas_call(
        _matvec_body,
        grid=(pl.cdiv(n, block_rows),),
        in_specs=[
            pl.BlockSpec((block_rows, D), lambda i: (i, 0)),
            pl.BlockSpec((1, D), lambda i: (0, 0)),
        ],
        out_specs=pl.BlockSpec((block_rows,), lambda i: (i,)),
        out_shape=jax.ShapeDtypeStruct((n,), jnp.float32),
    )(x, q)


@functools.cache
def _make_sc_scatter():
    mesh = plsc.VectorSubcoreMesh(core_axis_name="c", subcore_axis_name="s", num_cores=1, num_subcores=1)

    @functools.partial(
        pl.kernel,
        out_type=jax.ShapeDtypeStruct((CS,), jnp.float32),
        mesh=mesh,
        scratch_types=[
            pltpu.VMEM((CS,), jnp.float32),
            pltpu.VMEM((B,), jnp.int32),
            pltpu.VMEM((B,), jnp.float32),
        ],
        compiler_params=pltpu.CompilerParams(needs_layout_passes=False),
    )
    def _sc_scatter(idx_hbm, sval_hbm, out_hbm, s_v, idx_v, val_v):
        cid = jax.lax.axis_index("c")
        sid = jax.lax.axis_index("s")

        @pl.when(jnp.logical_and(cid == 0, sid == 0))
        def _():
            pltpu.sync_copy(idx_hbm, idx_v)
            pltpu.sync_copy(sval_hbm, val_v)

            sent = jnp.full((LANES,), SENTINEL, jnp.float32)

            def fill(k, carry):
                s_v[pl.ds(k * LANES, LANES)] = sent
                return carry

            jax.lax.fori_loop(0, CS // LANES, fill, 0)

            def step(k, carry):
                iv = idx_v[pl.ds(k * LANES, LANES)]
                vv = val_v[pl.ds(k * LANES, LANES)]
                plsc.store_scatter(s_v, [iv], vv)
                return carry

            jax.lax.fori_loop(0, B // LANES, step, 0)
            pltpu.sync_copy(s_v, out_hbm)

    return _sc_scatter


def _reduce_body(s_ref, w_ref, o_ref):
    s = s_ref[...]
    wv = w_ref[...]
    m = jnp.where(wv > -2.0, wv, s)
    w = jnp.exp((-BETA) * (1.0 - m))
    f = w * m
    seg_row = jax.lax.broadcasted_iota(jnp.int32, (128, 4), 0) // S
    seg_col = jax.lax.broadcasted_iota(jnp.int32, (128, 4), 1)
    seg = (seg_row == seg_col).astype(jnp.float32)
    o_ref[...] = jax.lax.dot_general(
        f, seg, (((1,), (0,)), ((), ())), preferred_element_type=jnp.float32)


def _reduce(s2, w2):
    return pl.pallas_call(
        _reduce_body,
        out_shape=jax.ShapeDtypeStruct((CS // 128, 4), jnp.float32),
    )(s2, w2)


def kernel(mem, idx, val, img_feat):
    q = img_feat.astype(jnp.float32)
    s_val = _matvec(val, q, 2048)
    written = _make_sc_scatter()(idx.astype(jnp.int32),
                                 s_val)
    s_mem = _matvec(mem, q, 2048)
    logits4 = _reduce(s_mem.reshape(CS // 128, 128),
                      written.reshape(CS // 128, 128))
    return logits4.reshape(C)
